# baseline (speedup 1.0000x reference)
import functools

import jax
import jax.numpy as jnp
from jax.experimental import pallas as pl
from jax.experimental.pallas import tpu as pltpu

N = 5000
D = 12544
H = 1024
NC = 4
NB = 12
OW = 128

BN = 1000
BK = 896
NK = D // BK
NN = N // BN


def _body(x_ref, w1_ref, w2_ref, b1_ref, b2_ref, w34_ref, b34_ref,
          out_ref, acc_ref):
    k = pl.program_id(0)
    n = pl.program_id(1)

    xb = x_ref[...].astype(jnp.bfloat16)
    w1b = w1_ref[...].astype(jnp.bfloat16)
    part = jax.lax.dot_general(
        xb, w1b, (((1,), (0,)), ((), ())),
        preferred_element_type=jnp.float32)

    rows = pl.ds(n * BN, BN)

    @pl.when(k == 0)
    def _():
        acc_ref[rows, :] = part

    @pl.when(k > 0)
    def _():
        acc_ref[rows, :] += part

    @pl.when(k == NK - 1)
    def _():
        h1 = jnp.maximum(acc_ref[rows, :] + b1_ref[...], 0.0)
        h2 = jax.lax.dot_general(
            h1.astype(jnp.bfloat16), w2_ref[...].astype(jnp.bfloat16),
            (((1,), (0,)), ((), ())),
            preferred_element_type=jnp.float32) + b2_ref[...]
        h2 = jnp.maximum(h2, 0.0)
        o = jax.lax.dot_general(
            h2.astype(jnp.bfloat16), w34_ref[...].astype(jnp.bfloat16),
            (((1,), (0,)), ((), ())),
            preferred_element_type=jnp.float32) + b34_ref[...]
        col = jax.lax.broadcasted_iota(jnp.int32, o.shape, 1)
        is_cls = col < NC
        neg = jnp.where(is_cls, o, -1e30)
        m = jnp.max(neg, axis=1, keepdims=True)
        e = jnp.where(is_cls, jnp.exp(o - m), 0.0)
        s = jnp.sum(e, axis=1, keepdims=True)
        out_ref[...] = jnp.where(is_cls, e / s, o)


@functools.partial(jax.jit, static_argnums=())
def kernel(feature_vectors, W1, b1, W2, b2, W3, b3, W4, b4):
    f32 = jnp.float32
    W34 = jnp.zeros((H, OW), f32).at[:, :NC].set(W3).at[:, NC:NC + NB].set(W4)
    b34 = jnp.zeros((1, OW), f32).at[0, :NC].set(b3).at[0, NC:NC + NB].set(b4)

    out = pl.pallas_call(
        _body,
        grid=(NK, NN),
        in_specs=[
            pl.BlockSpec((BN, BK), lambda k, n: (n, k)),
            pl.BlockSpec((BK, H), lambda k, n: (k, 0)),
            pl.BlockSpec((H, H), lambda k, n: (0, 0)),
            pl.BlockSpec((1, H), lambda k, n: (0, 0)),
            pl.BlockSpec((1, H), lambda k, n: (0, 0)),
            pl.BlockSpec((H, OW), lambda k, n: (0, 0)),
            pl.BlockSpec((1, OW), lambda k, n: (0, 0)),
        ],
        out_specs=pl.BlockSpec((BN, OW), lambda k, n: (n, 0)),
        out_shape=jax.ShapeDtypeStruct((N, OW), f32),
        scratch_shapes=[pltpu.VMEM((N, H), f32)],
        compiler_params=pltpu.CompilerParams(
            dimension_semantics=("arbitrary", "arbitrary"),
        ),
    )(feature_vectors, W1, W2, b1.reshape(1, H), b2.reshape(1, H), W34, b34)

    return out[:, :NC], out[:, NC:NC + NB]


# speedup vs baseline: 1.1153x; 1.1153x over previous
import jax
import jax.numpy as jnp
from jax.experimental import pallas as pl
from jax.experimental.pallas import tpu as pltpu

N = 5000
D = 12544
H = 1024
NC = 4
NB = 12
OW = 128

BN = 200
NN = N // BN


def _body(x_ref, w1_ref, w2_ref, b1_ref, b2_ref, w34_ref, b34_ref, out_ref):
    xb = x_ref[...].astype(jnp.bfloat16)
    h1 = jax.lax.dot_general(
        xb, w1_ref[...], (((1,), (0,)), ((), ())),
        preferred_element_type=jnp.float32)
    h1 = jnp.maximum(h1 + b1_ref[...], 0.0)
    h2 = jax.lax.dot_general(
        h1.astype(jnp.bfloat16), w2_ref[...], (((1,), (0,)), ((), ())),
        preferred_element_type=jnp.float32) + b2_ref[...]
    h2 = jnp.maximum(h2, 0.0)
    o = jax.lax.dot_general(
        h2.astype(jnp.bfloat16), w34_ref[...], (((1,), (0,)), ((), ())),
        preferred_element_type=jnp.float32) + b34_ref[...]
    col = jax.lax.broadcasted_iota(jnp.int32, o.shape, 1)
    is_cls = col < NC
    neg = jnp.where(is_cls, o, -1e30)
    m = jnp.max(neg, axis=1, keepdims=True)
    e = jnp.where(is_cls, jnp.exp(o - m), 0.0)
    s = jnp.sum(e, axis=1, keepdims=True)
    out_ref[...] = jnp.where(is_cls, e / s, o)


def kernel(feature_vectors, W1, b1, W2, b2, W3, b3, W4, b4):
    f32, bf16 = jnp.float32, jnp.bfloat16
    W34 = jnp.zeros((H, OW), f32).at[:, :NC].set(W3).at[:, NC:NC + NB].set(W4)
    b34 = jnp.zeros((1, OW), f32).at[0, :NC].set(b3).at[0, NC:NC + NB].set(b4)

    out = pl.pallas_call(
        _body,
        grid=(NN,),
        in_specs=[
            pl.BlockSpec((BN, D), lambda n: (n, 0)),
            pl.BlockSpec((D, H), lambda n: (0, 0)),
            pl.BlockSpec((H, H), lambda n: (0, 0)),
            pl.BlockSpec((1, H), lambda n: (0, 0)),
            pl.BlockSpec((1, H), lambda n: (0, 0)),
            pl.BlockSpec((H, OW), lambda n: (0, 0)),
            pl.BlockSpec((1, OW), lambda n: (0, 0)),
        ],
        out_specs=pl.BlockSpec((BN, OW), lambda n: (n, 0)),
        out_shape=jax.ShapeDtypeStruct((N, OW), f32),
        compiler_params=pltpu.CompilerParams(
            dimension_semantics=("arbitrary",),
        ),
    )(feature_vectors, W1.astype(bf16), W2.astype(bf16),
      b1.reshape(1, H), b2.reshape(1, H), W34.astype(bf16), b34)

    return out[:, :NC], out[:, NC:NC + NB]
